# baseline (speedup 1.0000x reference)
import numpy as np
import jax
import jax.numpy as jnp
from jax.experimental import pallas as pl
from jax.experimental.pallas import tpu as pltpu

_N = 262144
_BLK = 131072


def _offcol():
    corners = [(i, j, k) for i in (0, 1) for j in (0, 1) for k in (0, 1)]
    off = np.zeros((24, 1), np.float32)
    for d in range(3):
        for c in range(8):
            off[d * 8 + c, 0] = corners[c][d]
    return off


_OFF_NP = _offcol()


def _body(xt_ref, minb_ref, rng_ref, nl_ref, off_ref, xnt_ref, box_ref):
    xt = xt_ref[...]
    xn = (xt - minb_ref[...]) / rng_ref[...]
    xnt_ref[...] = xn
    mb = jnp.floor(xn * nl_ref[0, 0])
    for d in range(3):
        row = jnp.broadcast_to(mb[d:d + 1, :], (8, _BLK))
        box_ref[d * 8:(d + 1) * 8, :] = (row + off_ref[d * 8:(d + 1) * 8, :]
                                         ).astype(jnp.int32)


@jax.jit
def kernel(x, bounding_box, tables):
    del tables
    min_res = jnp.array([16.0], dtype=jnp.float32)
    max_res = jnp.array([512.0], dtype=jnp.float32)
    b = jnp.exp((jnp.log(max_res) - jnp.log(min_res)) / 15)
    nl = jnp.floor(min_res * b ** 15).reshape(1, 1)

    minb = bounding_box[0].reshape(3, 1)
    rng = (bounding_box[1] - bounding_box[0]).reshape(3, 1)
    off = jnp.asarray(_OFF_NP)

    xt = x.T
    grid = (_N // _BLK,)
    xnt, box24 = pl.pallas_call(
        _body,
        grid=grid,
        in_specs=[
            pl.BlockSpec((3, _BLK), lambda i: (0, i)),
            pl.BlockSpec((3, 1), lambda i: (0, 0)),
            pl.BlockSpec((3, 1), lambda i: (0, 0)),
            pl.BlockSpec((1, 1), lambda i: (0, 0)),
            pl.BlockSpec((24, 1), lambda i: (0, 0)),
        ],
        out_specs=[
            pl.BlockSpec((3, _BLK), lambda i: (0, i)),
            pl.BlockSpec((24, _BLK), lambda i: (0, i)),
        ],
        out_shape=[
            jax.ShapeDtypeStruct((3, _N), jnp.float32),
            jax.ShapeDtypeStruct((24, _N), jnp.int32),
        ],
    )(xt, minb, rng, nl, off)
    box = box24.reshape(3, 8, _N).transpose(2, 1, 0)
    return xnt.T, box


# speedup vs baseline: 1.1575x; 1.1575x over previous
import numpy as np
import jax
import jax.numpy as jnp
from jax.experimental import pallas as pl
from jax.experimental.pallas import tpu as pltpu

_N = 262144
_BLK = 65536


def _offcol():
    corners = [(i, j, k) for i in (0, 1) for j in (0, 1) for k in (0, 1)]
    off = np.zeros((24, 1), np.int32)
    for d in range(3):
        for c in range(8):
            off[d * 8 + c, 0] = corners[c][d]
    return off


_OFF_NP = _offcol()


def _body(xt_ref, minb_ref, rng_ref, nl_ref, off_ref, xnt_ref, box_ref):
    xt = xt_ref[...]
    xn = (xt - minb_ref[...]) / rng_ref[...]
    xnt_ref[...] = xn
    mbi = jnp.floor(xn * nl_ref[0, 0]).astype(jnp.int32)
    rows = jnp.concatenate(
        [jnp.broadcast_to(mbi[d:d + 1, :], (8, _BLK)) for d in range(3)],
        axis=0)
    box_ref[...] = rows + off_ref[...]


@jax.jit
def kernel(x, bounding_box, tables):
    del tables
    min_res = jnp.array([16.0], dtype=jnp.float32)
    max_res = jnp.array([512.0], dtype=jnp.float32)
    b = jnp.exp((jnp.log(max_res) - jnp.log(min_res)) / 15)
    nl = jnp.floor(min_res * b ** 15).reshape(1, 1)

    minb = bounding_box[0].reshape(3, 1)
    rng = (bounding_box[1] - bounding_box[0]).reshape(3, 1)
    off = jnp.asarray(_OFF_NP)

    xt = x.T
    grid = (_N // _BLK,)
    xnt, box24 = pl.pallas_call(
        _body,
        grid=grid,
        in_specs=[
            pl.BlockSpec((3, _BLK), lambda i: (0, i)),
            pl.BlockSpec((3, 1), lambda i: (0, 0)),
            pl.BlockSpec((3, 1), lambda i: (0, 0)),
            pl.BlockSpec((1, 1), lambda i: (0, 0)),
            pl.BlockSpec((24, 1), lambda i: (0, 0)),
        ],
        out_specs=[
            pl.BlockSpec((3, _BLK), lambda i: (0, i)),
            pl.BlockSpec((24, _BLK), lambda i: (0, i)),
        ],
        out_shape=[
            jax.ShapeDtypeStruct((3, _N), jnp.float32),
            jax.ShapeDtypeStruct((24, _N), jnp.int32),
        ],
    )(xt, minb, rng, nl, off)
    box = box24.reshape(3, 8, _N).transpose(2, 1, 0)
    return xnt.T, box
